# baseline (speedup 1.0000x reference)
import functools

import jax
import jax.numpy as jnp
from jax import lax
from jax.experimental import pallas as pl
from jax.experimental.pallas import tpu as pltpu
from jax.experimental.pallas import tpu_sc as plsc

VOCAB = 100000
D_MODEL = 1024
BATCH = 4
SEQ = 2048

_INFO = plsc.get_sparse_core_info()
_NC = _INFO.num_cores
_NS = _INFO.num_subcores
_NW = _NC * _NS
_B = BATCH * SEQ
_BPW = _B // _NW
_C = 32
_NCHUNK = _BPW // _C
_L = 16
_DSL = D_MODEL // _L


def _emb_body(x_hbm, tab_hbm, pe_hbm, out_hbm, idx_v, buf, pbuf, sem):
    wid = lax.axis_index("s") * _NC + lax.axis_index("c")
    base = wid * _BPW
    s_base = (wid % (SEQ // _BPW)) * _BPW
    pltpu.sync_copy(x_hbm.at[pl.ds(base, _BPW)], idx_v)
    for c in range(_NCHUNK):
        off = c * _C
        pltpu.sync_copy(pe_hbm.at[pl.ds(s_base + off, _C)], pbuf)
        pltpu.async_copy(tab_hbm.at[idx_v.at[pl.ds(off, _C)]], buf, sem).wait()

        def row_add(r, _):
            for k in range(_DSL):
                sl = pl.ds(k * _L, _L)
                buf[r, sl] = buf[r, sl] + pbuf[r, sl]
            return ()

        lax.fori_loop(0, _C, row_add, ())
        pltpu.sync_copy(buf, out_hbm.at[pl.ds(base + off, _C)])


@jax.jit
def _emb(x_flat, tok_table, pe):
    mesh = plsc.VectorSubcoreMesh(core_axis_name="c", subcore_axis_name="s")
    k = pl.kernel(
        _emb_body,
        out_type=jax.ShapeDtypeStruct((_B, D_MODEL), jnp.float32),
        mesh=mesh,
        scratch_types=[
            pltpu.VMEM((_BPW,), jnp.int32),
            pltpu.VMEM((_C, D_MODEL), jnp.float32),
            pltpu.VMEM((_C, D_MODEL), jnp.float32),
            pltpu.SemaphoreType.DMA,
        ],
    )
    return k(x_flat, tok_table, pe)


def kernel(x, tok_table, pe):
    out = _emb(x.reshape(-1), tok_table, pe)
    return out.reshape(BATCH, SEQ, D_MODEL)


# speedup vs baseline: 1.5008x; 1.5008x over previous
import functools

import jax
import jax.numpy as jnp
from jax import lax
from jax.experimental import pallas as pl
from jax.experimental.pallas import tpu as pltpu
from jax.experimental.pallas import tpu_sc as plsc

VOCAB = 100000
D_MODEL = 1024
BATCH = 4
SEQ = 2048

_INFO = plsc.get_sparse_core_info()
_NC = _INFO.num_cores
_NS = _INFO.num_subcores
_NW = _NC * _NS
_SPW = SEQ // _NW
_C = 32
_NH = _SPW // _C
_NCHUNK = _NH * BATCH
_L = 16
_DSL = D_MODEL // _L


def _emb_body(x_hbm, tab_hbm, pe_hbm, out_hbm,
              idx_v, buf0, buf1, pbuf, gsem, ssem, isem):
    wid = lax.axis_index("s") * _NC + lax.axis_index("c")
    s0 = wid * _SPW
    bufs = (buf0, buf1)

    idx_copies = [
        pltpu.async_copy(x_hbm.at[pl.ds(b * SEQ + s0, _SPW)],
                         idx_v.at[pl.ds(b * _SPW, _SPW)], isem)
        for b in range(BATCH)
    ]
    for cp in idx_copies:
        cp.wait()

    def chunk_hb(i):
        return i // BATCH, i % BATCH

    def issue_gather(i):
        h, b = chunk_hb(i)
        return pltpu.async_copy(
            tab_hbm.at[idx_v.at[pl.ds(b * _SPW + h * _C, _C)]],
            bufs[i % 2], gsem)

    def issue_store(i):
        h, b = chunk_hb(i)
        return pltpu.async_copy(
            bufs[i % 2],
            out_hbm.at[pl.ds(b * SEQ + s0 + h * _C, _C)], ssem)

    stores = [None] * _NCHUNK
    g_next = issue_gather(0)
    for i in range(_NCHUNK):
        h, b = chunk_hb(i)
        if b == 0:
            pltpu.sync_copy(pe_hbm.at[pl.ds(s0 + h * _C, _C)], pbuf)
        g_cur = g_next
        if i + 1 < _NCHUNK:
            if i >= 1:
                stores[i - 1].wait()
            g_next = issue_gather(i + 1)
        g_cur.wait()

        buf = bufs[i % 2]

        def row_add(r, _):
            for k in range(_DSL):
                sl = pl.ds(k * _L, _L)
                buf[r, sl] = buf[r, sl] + pbuf[r, sl]
            return ()

        lax.fori_loop(0, _C, row_add, ())
        stores[i] = issue_store(i)
    stores[_NCHUNK - 2].wait()
    stores[_NCHUNK - 1].wait()


@jax.jit
def _emb(x_flat, tok_table, pe):
    mesh = plsc.VectorSubcoreMesh(core_axis_name="c", subcore_axis_name="s")
    k = pl.kernel(
        _emb_body,
        out_type=jax.ShapeDtypeStruct((BATCH * SEQ, D_MODEL), jnp.float32),
        mesh=mesh,
        scratch_types=[
            pltpu.VMEM((BATCH * _SPW,), jnp.int32),
            pltpu.VMEM((_C, D_MODEL), jnp.float32),
            pltpu.VMEM((_C, D_MODEL), jnp.float32),
            pltpu.VMEM((_C, D_MODEL), jnp.float32),
            pltpu.SemaphoreType.DMA,
            pltpu.SemaphoreType.DMA,
            pltpu.SemaphoreType.DMA,
        ],
    )
    return k(x_flat, tok_table, pe)


def kernel(x, tok_table, pe):
    out = _emb(x.reshape(-1), tok_table, pe)
    return out.reshape(BATCH, SEQ, D_MODEL)
